# baseline (speedup 1.0000x reference)
import jax
import jax.numpy as jnp
from jax import lax
from jax.experimental import pallas as pl
from jax.experimental.pallas import tpu as pltpu
from jax.experimental.pallas import tpu_sc as plsc

VOCAB = 100000
HIDDEN = 64
MAX_POS = 200
BATCH = 4096
SEQ = 200
EPS = 1e-5

NC = 2
NS = 16
NW = NC * NS
BATCH_PER_W = BATCH // NW
CHUNK = SEQ
NCHUNK = BATCH_PER_W
NBUF = 4

_RSQRT_MAGIC = 0x5F3759DF


def _sc_body(ids_hbm, wt_hbm, pt_hbm, gm_hbm, bt_hbm, out_hbm,
             idx_v, pos_v, g_v, b_v,
             buf0, buf1, buf2, buf3,
             gs0, gs1, gs2, gs3, ss0, ss1, ss2, ss3):
  wid = lax.axis_index("s") * NC + lax.axis_index("c")
  base_b = wid * BATCH_PER_W

  pltpu.sync_copy(ids_hbm.at[pl.ds(base_b, BATCH_PER_W), :], idx_v)
  pltpu.sync_copy(pt_hbm, pos_v)
  pltpu.sync_copy(gm_hbm, g_v)
  pltpu.sync_copy(bt_hbm, b_v)

  bufs = [buf0, buf1, buf2, buf3]
  gsems = [gs0, gs1, gs2, gs3]
  ssems = [ss0, ss1, ss2, ss3]

  gvs = [g_v[pl.ds(16 * j, 16)] for j in range(4)]
  bvs = [b_v[pl.ds(16 * j, 16)] for j in range(4)]
  magic = jnp.full((16,), _RSQRT_MAGIC, dtype=jnp.int32)
  half = jnp.float32(0.5)
  three_half = jnp.full((16,), 1.5, dtype=jnp.float32)

  def compute(buf, row_unroll=1, punroll=8):
    @plsc.parallel_loop(0, CHUNK // row_unroll, unroll=punroll)
    def row_body(ri):
      for u in range(row_unroll):
        r = ri * row_unroll + u
        xs = []
        for j in range(4):
          w = buf[r, pl.ds(16 * j, 16)]
          p = pos_v[r, pl.ds(16 * j, 16)]
          xs.append(w + p)
        s = (xs[0] + xs[1]) + (xs[2] + xs[3])
        sq = (xs[0] * xs[0] + xs[1] * xs[1]) + (xs[2] * xs[2] + xs[3] * xs[3])
        mean = jnp.sum(s) * jnp.float32(1.0 / HIDDEN)
        var = jnp.sum(sq) * jnp.float32(1.0 / HIDDEN) - mean * mean
        mv = jnp.full((16,), mean, dtype=jnp.float32)
        av = jnp.full((16,), var + jnp.float32(EPS), dtype=jnp.float32)
        iv = lax.bitcast_convert_type(av, jnp.int32)
        iv = magic - lax.shift_right_logical(iv, 1)
        y = lax.bitcast_convert_type(iv, jnp.float32)
        ah = av * half
        for _ in range(2):
          y = y * (three_half - ah * y * y)
        for j in range(4):
          buf[r, pl.ds(16 * j, 16)] = (xs[j] - mv) * y * gvs[j] + bvs[j]

  def start_gather(c, b):
    pltpu.async_copy(wt_hbm.at[idx_v.at[c]], bufs[b], gsems[b])

  def wait_gather(c, b):
    pltpu.make_async_copy(wt_hbm.at[idx_v.at[c]], bufs[b], gsems[b]).wait()

  def start_scatter(c, b):
    pltpu.async_copy(
        bufs[b], out_hbm.at[base_b + c, :, pl.ds(0, HIDDEN)], ssems[b])

  def wait_scatter(c, b):
    pltpu.make_async_copy(
        bufs[b], out_hbm.at[base_b + c, :, pl.ds(0, HIDDEN)], ssems[b]).wait()

  start_gather(0, 0)
  start_gather(1, 1)

  for c0 in (0, 1):
    b = c0 % NBUF
    wait_gather(c0, b)
    start_gather(c0 + 2, (b + 2) % NBUF)
    compute(bufs[b], row_unroll=2, punroll=1)
    start_scatter(c0, b)

  def outer(i, _):
    for u in range(NBUF):
      c = 2 + i * NBUF + u
      b = (2 + u) % NBUF
      pf = (b + 2) % NBUF
      wait_gather(c, b)
      wait_scatter(c - 2, pf)
      start_gather(c + 2, pf)
      compute(bufs[b])
      start_scatter(c, b)
    return 0

  lax.fori_loop(0, (NCHUNK - 4) // NBUF, outer, 0, unroll=False)

  for c0 in (NCHUNK - 2, NCHUNK - 1):
    b = c0 % NBUF
    wait_gather(c0, b)
    wait_scatter(c0 - 2, (b + 2) % NBUF)
    compute(bufs[b], row_unroll=2, punroll=1)
    start_scatter(c0, b)

  wait_scatter(NCHUNK - 2, (NCHUNK - 2) % NBUF)
  wait_scatter(NCHUNK - 1, (NCHUNK - 1) % NBUF)


@jax.jit
def _sc_embed(input_id, word_table, pos_table, ln_gamma, ln_beta):
  mesh = plsc.VectorSubcoreMesh(core_axis_name="c", subcore_axis_name="s")
  f = pl.kernel(
      _sc_body,
      out_type=jax.ShapeDtypeStruct((BATCH, SEQ, 128), jnp.float32),
      mesh=mesh,
      compiler_params=pltpu.CompilerParams(
          needs_layout_passes=False, use_tc_tiling_on_sc=False),
      scratch_types=[
          pltpu.VMEM((NCHUNK, SEQ), jnp.int32),
          pltpu.VMEM((MAX_POS, HIDDEN), jnp.float32),
          pltpu.VMEM((HIDDEN,), jnp.float32),
          pltpu.VMEM((HIDDEN,), jnp.float32),
          pltpu.VMEM((CHUNK, HIDDEN), jnp.float32),
          pltpu.VMEM((CHUNK, HIDDEN), jnp.float32),
          pltpu.VMEM((CHUNK, HIDDEN), jnp.float32),
          pltpu.VMEM((CHUNK, HIDDEN), jnp.float32),
          pltpu.SemaphoreType.DMA,
          pltpu.SemaphoreType.DMA,
          pltpu.SemaphoreType.DMA,
          pltpu.SemaphoreType.DMA,
          pltpu.SemaphoreType.DMA,
          pltpu.SemaphoreType.DMA,
          pltpu.SemaphoreType.DMA,
          pltpu.SemaphoreType.DMA,
      ],
  )
  return f(input_id, word_table, pos_table, ln_gamma, ln_beta)


_SLICE_BLK = 1024


def _tc_slice_body(i_ref, o_ref):
  o_ref[...] = i_ref[:, :HIDDEN]


def _tc_slice(x):
  flat = x.reshape(BATCH * SEQ, 128)
  out = pl.pallas_call(
      _tc_slice_body,
      grid=(BATCH * SEQ // _SLICE_BLK,),
      in_specs=[pl.BlockSpec((_SLICE_BLK, 128), lambda i: (i, 0))],
      out_specs=pl.BlockSpec((_SLICE_BLK, HIDDEN), lambda i: (i, 0)),
      out_shape=jax.ShapeDtypeStruct((BATCH * SEQ, HIDDEN), jnp.float32),
  )(flat)
  return out.reshape(BATCH, SEQ, HIDDEN)


def kernel(input_id, word_table, pos_table, ln_gamma, ln_beta):
  out = _sc_embed(input_id, word_table, pos_table, ln_gamma, ln_beta)
  return _tc_slice(out)


# speedup vs baseline: 2.0816x; 2.0816x over previous
import jax
import jax.numpy as jnp
from jax import lax
from jax.experimental import pallas as pl
from jax.experimental.pallas import tpu as pltpu
from jax.experimental.pallas import tpu_sc as plsc

VOCAB = 100000
HIDDEN = 64
MAX_POS = 200
BATCH = 4096
SEQ = 200
EPS = 1e-5

NC = 2
NS = 16
NW = NC * NS
BATCH_PER_W = BATCH // NW
CHUNK = SEQ
NCHUNK = BATCH_PER_W
NBUF = 4

_RSQRT_MAGIC = 0x5F3759DF


def _sc_body(ids_hbm, wt_hbm, pt_hbm, gm_hbm, bt_hbm, out_hbm,
             idx_v, pos_v, g_v, b_v,
             buf0, buf1, buf2, buf3,
             gs0, gs1, gs2, gs3, ss0, ss1, ss2, ss3):
  wid = lax.axis_index("s") * NC + lax.axis_index("c")
  base_b = wid * BATCH_PER_W

  pltpu.sync_copy(ids_hbm.at[pl.ds(base_b, BATCH_PER_W), :], idx_v)
  pltpu.sync_copy(pt_hbm, pos_v)
  pltpu.sync_copy(gm_hbm, g_v)
  pltpu.sync_copy(bt_hbm, b_v)

  bufs = [buf0, buf1, buf2, buf3]
  gsems = [gs0, gs1, gs2, gs3]
  ssems = [ss0, ss1, ss2, ss3]

  gvs = [g_v[pl.ds(16 * j, 16)] for j in range(4)]
  bvs = [b_v[pl.ds(16 * j, 16)] for j in range(4)]
  magic = jnp.full((16,), _RSQRT_MAGIC, dtype=jnp.int32)
  half = jnp.float32(0.5)
  three_half = jnp.full((16,), 1.5, dtype=jnp.float32)

  def compute(buf, row_unroll=1, punroll=8):
    @plsc.parallel_loop(0, CHUNK // row_unroll, unroll=punroll)
    def row_body(ri):
      for u in range(row_unroll):
        r = ri * row_unroll + u
        xs = []
        for j in range(4):
          w = buf[r, pl.ds(16 * j, 16)]
          p = pos_v[r, pl.ds(16 * j, 16)]
          xs.append(w + p)
        s = (xs[0] + xs[1]) + (xs[2] + xs[3])
        sq = (xs[0] * xs[0] + xs[1] * xs[1]) + (xs[2] * xs[2] + xs[3] * xs[3])
        mean = jnp.sum(s) * jnp.float32(1.0 / HIDDEN)
        var = jnp.sum(sq) * jnp.float32(1.0 / HIDDEN) - mean * mean
        mv = jnp.full((16,), mean, dtype=jnp.float32)
        av = jnp.full((16,), var + jnp.float32(EPS), dtype=jnp.float32)
        iv = lax.bitcast_convert_type(av, jnp.int32)
        iv = magic - lax.shift_right_logical(iv, 1)
        y = lax.bitcast_convert_type(iv, jnp.float32)
        ah = av * half
        for _ in range(2):
          y = y * (three_half - ah * y * y)
        for j in range(4):
          buf[r, pl.ds(16 * j, 16)] = (xs[j] - mv) * y * gvs[j] + bvs[j]

  def start_gather(c, b):
    pltpu.async_copy(wt_hbm.at[idx_v.at[c]], bufs[b], gsems[b])

  def wait_gather(c, b):
    pltpu.make_async_copy(wt_hbm.at[idx_v.at[c]], bufs[b], gsems[b]).wait()

  def start_scatter(c, b):
    pltpu.async_copy(
        bufs[b], out_hbm.at[base_b + c, :, pl.ds(0, HIDDEN)], ssems[b])

  def wait_scatter(c, b):
    pltpu.make_async_copy(
        bufs[b], out_hbm.at[base_b + c, :, pl.ds(0, HIDDEN)], ssems[b]).wait()

  start_gather(0, 0)
  start_gather(1, 1)

  for c0 in (0, 1):
    b = c0 % NBUF
    wait_gather(c0, b)
    start_gather(c0 + 2, (b + 2) % NBUF)
    compute(bufs[b], row_unroll=2, punroll=1)
    start_scatter(c0, b)

  def outer(i, _):
    for u in range(NBUF):
      c = 2 + i * NBUF + u
      b = (2 + u) % NBUF
      pf = (b + 2) % NBUF
      wait_gather(c, b)
      wait_scatter(c - 2, pf)
      start_gather(c + 2, pf)
      compute(bufs[b])
      start_scatter(c, b)
    return 0

  lax.fori_loop(0, (NCHUNK - 4) // NBUF, outer, 0, unroll=False)

  for c0 in (NCHUNK - 2, NCHUNK - 1):
    b = c0 % NBUF
    wait_gather(c0, b)
    wait_scatter(c0 - 2, (b + 2) % NBUF)
    compute(bufs[b], row_unroll=2, punroll=1)
    start_scatter(c0, b)

  wait_scatter(NCHUNK - 2, (NCHUNK - 2) % NBUF)
  wait_scatter(NCHUNK - 1, (NCHUNK - 1) % NBUF)


@jax.jit
def _sc_embed(input_id, word_table, pos_table, ln_gamma, ln_beta):
  mesh = plsc.VectorSubcoreMesh(core_axis_name="c", subcore_axis_name="s")
  f = pl.kernel(
      _sc_body,
      out_type=jax.ShapeDtypeStruct((BATCH, SEQ, 128), jnp.float32),
      mesh=mesh,
      compiler_params=pltpu.CompilerParams(
          needs_layout_passes=False, use_tc_tiling_on_sc=False),
      scratch_types=[
          pltpu.VMEM((NCHUNK, SEQ), jnp.int32),
          pltpu.VMEM((MAX_POS, HIDDEN), jnp.float32),
          pltpu.VMEM((HIDDEN,), jnp.float32),
          pltpu.VMEM((HIDDEN,), jnp.float32),
          pltpu.VMEM((CHUNK, HIDDEN), jnp.float32),
          pltpu.VMEM((CHUNK, HIDDEN), jnp.float32),
          pltpu.VMEM((CHUNK, HIDDEN), jnp.float32),
          pltpu.VMEM((CHUNK, HIDDEN), jnp.float32),
          pltpu.SemaphoreType.DMA,
          pltpu.SemaphoreType.DMA,
          pltpu.SemaphoreType.DMA,
          pltpu.SemaphoreType.DMA,
          pltpu.SemaphoreType.DMA,
          pltpu.SemaphoreType.DMA,
          pltpu.SemaphoreType.DMA,
          pltpu.SemaphoreType.DMA,
      ],
  )
  return f(input_id, word_table, pos_table, ln_gamma, ln_beta)


def kernel(input_id, word_table, pos_table, ln_gamma, ln_beta):
  out = _sc_embed(input_id, word_table, pos_table, ln_gamma, ln_beta)
  return out[:, :, :HIDDEN]
